# baseline (speedup 1.0000x reference)
import jax
import jax.numpy as jnp
from jax import lax
from jax.experimental import pallas as pl
from jax.experimental.pallas import tpu as pltpu
from jax.experimental.pallas import tpu_sc as plsc

_VOCAB = 100000
_MAX_LEN = 200
_D = 64
_BATCH = 4096

_NC = 2
_NS = 16
_NW = _NC * _NS

_BT = _BATCH // 128
_QQ = _BT // 2
_N_UNITS = _MAX_LEN * _QQ
_PER_W = _N_UNITS // _NW
_NBUF = 2
_N_ROUNDS = _PER_W // _NBUF


def _body(xf_ref, tok_ref, pos_ref, out_ref, idx_v, rows_v, out_v, pos_v,
          *sems):
    in_sems = sems[:_NBUF]
    out_sems = sems[_NBUF:]
    wid = lax.axis_index("c") * _NS + lax.axis_index("s")
    u0 = wid * _PER_W

    pltpu.sync_copy(xf_ref.at[pl.ds(u0 * 2, _PER_W * 2)], idx_v)
    pltpu.sync_copy(pos_ref, pos_v)

    def gathers_start(i, s):
        for j in range(2):
            pltpu.async_copy(
                tok_ref.at[idx_v.at[i * 2 + j]], rows_v.at[s, j], in_sems[s])

    def gathers_wait(i, s):
        for j in range(2):
            pltpu.make_async_copy(
                tok_ref.at[idx_v.at[i * 2 + j]], rows_v.at[s, j],
                in_sems[s]).wait()

    def stores_start(t, qq, s):
        for dt in range(8):
            pltpu.async_copy(
                out_v.at[s, dt], out_ref.at[t, dt, pl.ds(qq * 16, 16)],
                out_sems[s])

    def stores_wait(s):
        pltpu.make_async_copy(
            out_v.at[s], out_ref.at[0, :, pl.ds(0, 16)], out_sems[s]).wait()

    iota16 = lax.iota(jnp.int32, 16)

    gathers_start(0, 0)

    def round_body(r, _):
        for s in range(_NBUF):
            i = r * _NBUF + s
            u = u0 + i
            t = u // _QQ
            qq = u % _QQ

            @pl.when(i + 1 < _PER_W)
            def _():
                gathers_start(i + 1, 1 - s)

            gathers_wait(i, s)

            @pl.when(i >= _NBUF)
            def _():
                stores_wait(s)

            pv = [pos_v[t, pl.ds(k * 16, 16)] for k in range(_D // 16)]

            @plsc.parallel_loop(0, 8, step=1, unroll=1)
            def c_loop(c):
                row_idx = iota16 + c * 16
                for j in range(2):
                    for d in range(_D):
                        dt, dr = d // 8, d % 8
                        vals = plsc.load_gather(
                            rows_v.at[s, j],
                            [row_idx, jnp.full((16,), d, jnp.int32)])
                        pos_vec = jnp.full((16,), pv[d // 16][d % 16],
                                           jnp.float32)
                        out_v[s, dt, j * 8 + dr, pl.ds(c * 16, 16)] = (
                            vals + pos_vec)

            stores_start(t, qq, s)
        return 0

    lax.fori_loop(0, _N_ROUNDS, round_body, 0)

    stores_wait(0)
    stores_wait(1)


@jax.jit
def kernel(x, token_table, pos_table):
    xf = x.astype(jnp.int32).T.reshape(_MAX_LEN * _BT, 128)
    mesh = plsc.VectorSubcoreMesh(core_axis_name="c", subcore_axis_name="s")
    out = pl.kernel(
        _body,
        out_type=jax.ShapeDtypeStruct((_MAX_LEN, 8, 8 * _BT, 128),
                                      jnp.float32),
        mesh=mesh,
        scratch_types=[
            pltpu.VMEM((_PER_W * 2, 128), jnp.int32),
            pltpu.VMEM((_NBUF, 2, 128, _D), jnp.float32),
            pltpu.VMEM((_NBUF, 8, 16, 128), jnp.float32),
            pltpu.VMEM((_MAX_LEN, _D), jnp.float32),
        ] + [pltpu.SemaphoreType.DMA] * (2 * _NBUF),
        compiler_params=pltpu.CompilerParams(use_tc_tiling_on_sc=False,
                                             needs_layout_passes=False),
    )(xf, token_table, pos_table)
    p5 = out.reshape(_MAX_LEN, 8, _BT, 8, 128)
    return p5.transpose(2, 4, 0, 1, 3).reshape(_BATCH, _MAX_LEN, _D)


# speedup vs baseline: 1.6332x; 1.6332x over previous
import jax
import jax.numpy as jnp
from jax import lax
from jax.experimental import pallas as pl
from jax.experimental.pallas import tpu as pltpu
from jax.experimental.pallas import tpu_sc as plsc

_VOCAB = 100000
_MAX_LEN = 200
_D = 64
_BATCH = 4096

_NC = 2
_NS = 16
_NW = _NC * _NS

_BT = _BATCH // 128
_QQ = _BT // 2
_N_UNITS = _MAX_LEN * _QQ
_PER_W = _N_UNITS // _NW
_NBUF = 2
_N_ROUNDS = _PER_W // _NBUF


def _body(xf_ref, tok_ref, pos_ref, out_ref, idx_v, rows_v, pos_v, *sems):
    in_sems = sems[:_NBUF]
    out_sems = sems[_NBUF:]
    wid = lax.axis_index("c") * _NS + lax.axis_index("s")
    u0 = wid * _PER_W

    pltpu.sync_copy(xf_ref.at[pl.ds(u0 * 2, _PER_W * 2)], idx_v)
    pltpu.sync_copy(pos_ref, pos_v)

    def gathers_start(i, s):
        for j in range(2):
            pltpu.async_copy(
                tok_ref.at[idx_v.at[i * 2 + j]],
                rows_v.at[s, pl.ds(j * 128, 128)], in_sems[s])

    def gathers_wait(i, s):
        for j in range(2):
            pltpu.make_async_copy(
                tok_ref.at[idx_v.at[i * 2 + j]],
                rows_v.at[s, pl.ds(j * 128, 128)], in_sems[s]).wait()

    def store_start(t, qq, s):
        pltpu.async_copy(
            rows_v.at[s], out_ref.at[t, pl.ds(qq * 256, 256)], out_sems[s])

    def store_wait(s):
        pltpu.make_async_copy(
            rows_v.at[s], out_ref.at[0, pl.ds(0, 256)], out_sems[s]).wait()

    gathers_start(0, 0)

    def round_body(r, _):
        for s in range(_NBUF):
            i = r * _NBUF + s
            u = u0 + i
            t = u // _QQ
            qq = u % _QQ

            @pl.when(i >= 1)
            def _():
                store_wait(1 - s)

            @pl.when(i + 1 < _PER_W)
            def _():
                gathers_start(i + 1, 1 - s)

            gathers_wait(i, s)

            pv = [pos_v[t, pl.ds(k * 16, 16)] for k in range(_D // 16)]

            @plsc.parallel_loop(0, 256, step=1, unroll=4)
            def row_loop(row):
                for k in range(_D // 16):
                    sl = pl.ds(k * 16, 16)
                    rows_v[s, row, sl] += pv[k]

            store_start(t, qq, s)
        return 0

    lax.fori_loop(0, _N_ROUNDS, round_body, 0)

    store_wait((_PER_W - 1) % _NBUF)


@jax.jit
def kernel(x, token_table, pos_table):
    xf = x.astype(jnp.int32).T.reshape(_MAX_LEN * _BT, 128)
    mesh = plsc.VectorSubcoreMesh(core_axis_name="c", subcore_axis_name="s")
    out = pl.kernel(
        _body,
        out_type=jax.ShapeDtypeStruct((_MAX_LEN, _BATCH, _D), jnp.float32),
        mesh=mesh,
        scratch_types=[
            pltpu.VMEM((_PER_W * 2, 128), jnp.int32),
            pltpu.VMEM((_NBUF, 256, _D), jnp.float32),
            pltpu.VMEM((_MAX_LEN, _D), jnp.float32),
        ] + [pltpu.SemaphoreType.DMA] * (2 * _NBUF),
        compiler_params=pltpu.CompilerParams(use_tc_tiling_on_sc=False),
    )(xf, token_table, pos_table)
    return out.transpose(1, 0, 2)
